# baseline (speedup 1.0000x reference)
import functools

import jax
import jax.numpy as jnp
from jax import lax
from jax.experimental import pallas as pl
from jax.experimental.pallas import tpu as pltpu
from jax.experimental.pallas import tpu_sc as plsc

NC = 2
NS = 16
NW = NC * NS
L = 16

B = 128


def _build(N, S, H, W, C):
    P = N * S
    HW = H * W
    PPW = P // NW
    NCHUNK = PPW // B
    UV = ((1, 2), (0, 2), (0, 1))

    mesh = plsc.VectorSubcoreMesh(
        core_axis_name="c", subcore_axis_name="s",
        num_cores=NC, num_subcores=NS)

    NT = C // 8
    SCH = S // 128

    @functools.partial(
        pl.kernel,
        out_type=jax.ShapeDtypeStruct((N * 3 * NT, SCH, 8 * 128), jnp.float32),
        mesh=mesh,
        compiler_params=pltpu.CompilerParams(
            needs_layout_passes=False, use_tc_tiling_on_sc=False),
        scratch_types=(
            [pltpu.VMEM((B,), jnp.float32)] * 3
            + [pltpu.VMEM((B,), jnp.int32)] * 12
            + [pltpu.VMEM((12 * B,), jnp.float32)]
            + [pltpu.VMEM((B, C), jnp.float32)] * 12
            + [pltpu.VMEM((3 * NT, 8 * 128), jnp.float32)]
            + [pltpu.SemaphoreType.DMA]
        ),
    )
    def tri(xt_hbm, table_hbm, out_hbm, *scr):
        c_refs = scr[0:3]
        i_refs = scr[3:15]
        w_v = scr[15]
        r_refs = scr[16:28]
        o_v = scr[28]
        sem = scr[29]
        table2 = table_hbm

        wid = lax.axis_index("c") * NS + lax.axis_index("s")
        wbase = wid * PPW
        n_id = wbase // S
        s0 = wbase - n_id * S

        def chunk_body(t, carry):
            base = wbase + t * B
            for j in range(3):
                pltpu.sync_copy(xt_hbm.at[pl.ds(j * P + base, B)], c_refs[j])

            cps = []
            for q in range(3):
                uj, vj = UV[q]

                def iw_body(i, carry2, q=q, uj=uj, vj=vj):
                    s = pl.ds(i * L, L)
                    u = c_refs[uj][s]
                    v = c_refs[vj][s]
                    gu = (u + 1.0) * (0.5 * (W - 1))
                    gv = (v + 1.0) * (0.5 * (H - 1))
                    u0 = gu.astype(jnp.int32)
                    v0 = gv.astype(jnp.int32)
                    wx = gu - u0.astype(jnp.float32)
                    wy = gv - v0.astype(jnp.float32)
                    du = jnp.minimum(u0 + 1, W - 1) - u0
                    dv = (jnp.minimum(v0 + 1, H - 1) - v0) * W
                    base00 = v0 * W + u0 + q * HW
                    i_refs[4 * q + 0][s] = base00
                    i_refs[4 * q + 1][s] = base00 + du
                    i_refs[4 * q + 2][s] = base00 + dv
                    i_refs[4 * q + 3][s] = base00 + dv + du
                    w_v[pl.ds((4 * q + 0) * B + i * L, L)] = (1.0 - wx) * (1.0 - wy)
                    w_v[pl.ds((4 * q + 1) * B + i * L, L)] = wx * (1.0 - wy)
                    w_v[pl.ds((4 * q + 2) * B + i * L, L)] = (1.0 - wx) * wy
                    w_v[pl.ds((4 * q + 3) * B + i * L, L)] = wx * wy
                    return carry2

                lax.fori_loop(0, B // L, iw_body, 0)
                cps.append([
                    pltpu.async_copy(
                        table2.at[i_refs[4 * q + k]], r_refs[4 * q + k], sem)
                    for k in range(4)])

            iota_l = lax.iota(jnp.int32, L)
            for q in range(3):
                for cp in cps[q]:
                    cp.wait()
                rr = [r_refs[4 * q + k] for k in range(4)]

                def fma_body(g, carry2, q=q, rr=rr):
                    w0 = w_v[pl.ds((4 * q + 0) * B + g * L, L)]
                    w1 = w_v[pl.ds((4 * q + 1) * B + g * L, L)]
                    w2 = w_v[pl.ds((4 * q + 2) * B + g * L, L)]
                    w3 = w_v[pl.ds((4 * q + 3) * B + g * L, L)]
                    row_i = iota_l + g * L
                    for c in range(C):
                        col_i = jnp.full((L,), c, dtype=jnp.int32)
                        a0 = plsc.load_gather(rr[0], [row_i, col_i])
                        a1 = plsc.load_gather(rr[1], [row_i, col_i])
                        a2 = plsc.load_gather(rr[2], [row_i, col_i])
                        a3 = plsc.load_gather(rr[3], [row_i, col_i])
                        o_v[q * NT + c // 8,
                            pl.ds((c % 8) * 128 + g * L, L)] = (
                            w0 * a0 + w1 * a1 + w2 * a2 + w3 * a3)
                    return carry2

                lax.fori_loop(0, B // L, fma_body, 0)

            pltpu.sync_copy(
                o_v,
                out_hbm.at[pl.ds(n_id * 3 * NT, 3 * NT), (s0 + t * B) // 128, :])
            return carry

        lax.fori_loop(0, NCHUNK, chunk_body, 0)

    return tri


def kernel(x, fmx, fmy, fmz):
    N, S, _ = x.shape
    C = fmx.shape[1]
    H, W = fmx.shape[2], fmx.shape[3]
    P = N * S

    planes = jnp.stack([fmx[0], fmy[0], fmz[0]], axis=0)
    table = planes.transpose(0, 2, 3, 1).reshape(3 * H * W, C)
    xt = x.reshape(P, 3).T.reshape(3 * P)

    raw = _build(N, S, H, W, C)(xt, table)
    o = raw.reshape(N, 3, C // 8, S // 128, 8, 128)
    o = o.transpose(0, 3, 5, 1, 2, 4)
    return o.reshape(N, S, 3, C)


# speedup vs baseline: 3.2114x; 3.2114x over previous
import functools

import jax
import jax.numpy as jnp
from jax import lax
from jax.experimental import pallas as pl
from jax.experimental.pallas import tpu as pltpu
from jax.experimental.pallas import tpu_sc as plsc

NC = 2
NS = 16
NW = NC * NS
L = 16

B = 128


def _build(N, S, H, W, C):
    P = N * S
    HW = H * W
    PPW = P // NW
    NCHUNK = PPW // B
    UV = ((1, 2), (0, 2), (0, 1))

    mesh = plsc.VectorSubcoreMesh(
        core_axis_name="c", subcore_axis_name="s",
        num_cores=NC, num_subcores=NS)

    NT = C // 8
    SCH = S // 128

    buf = lambda shape, dt: [pltpu.VMEM(shape, dt)] * 2

    @functools.partial(
        pl.kernel,
        out_type=jax.ShapeDtypeStruct((N * 3 * NT, SCH, 8, 128), jnp.float32),
        mesh=mesh,
        compiler_params=pltpu.CompilerParams(
            needs_layout_passes=False, use_tc_tiling_on_sc=False),
        scratch_types=(
            buf((3, B), jnp.float32)
            + buf((12, B), jnp.int32)
            + buf((12 * B,), jnp.float32)
            + [pltpu.VMEM((B, C), jnp.float32)] * 24
            + [pltpu.VMEM((C * L,), jnp.int32)] * 2
            + [pltpu.VMEM((3 * NT, 8, 129), jnp.float32)]
            + [pltpu.SemaphoreType.DMA] * 2
        ),
    )
    def tri(xt_hbm, table_hbm, out_hbm, *scr):
        c_bufs = scr[0:2]
        i_bufs = scr[2:4]
        w_bufs = scr[4:6]
        r_bufs = (scr[6:18], scr[18:30])
        dc_v, sr_v = scr[30:32]
        o_v = scr[32]
        sems = scr[33:35]
        table2 = table_hbm
        iota_l = lax.iota(jnp.int32, L)

        dc_v[pl.ds(0, L)] = iota_l >> 3
        dc_v[pl.ds(L, L)] = (iota_l >> 3) + 2
        sr_v[pl.ds(0, L)] = iota_l & 7
        sr_v[pl.ds(L, L)] = iota_l & 7

        wid = lax.axis_index("c") * NS + lax.axis_index("s")
        wbase = wid * PPW
        n_id = wbase // S
        s0 = wbase - n_id * S

        def stage1(t, h):
            c_v, i_v, w_v, sem = c_bufs[h], i_bufs[h], w_bufs[h], sems[h]
            base = wbase + t * B
            for j in range(3):
                pltpu.sync_copy(xt_hbm.at[pl.ds(j * P + base, B)], c_v.at[j])
            for q in range(3):
                uj, vj = UV[q]

                def iw_body(i, carry2, q=q, uj=uj, vj=vj,
                            c_v=c_v, i_v=i_v, w_v=w_v):
                    s = pl.ds(i * L, L)
                    u = c_v[uj, s]
                    v = c_v[vj, s]
                    gu = (u + 1.0) * (0.5 * (W - 1))
                    gv = (v + 1.0) * (0.5 * (H - 1))
                    u0 = gu.astype(jnp.int32)
                    v0 = gv.astype(jnp.int32)
                    wx = gu - u0.astype(jnp.float32)
                    wy = gv - v0.astype(jnp.float32)
                    du = jnp.minimum(u0 + 1, W - 1) - u0
                    dv = (jnp.minimum(v0 + 1, H - 1) - v0) * W
                    base00 = v0 * W + u0 + q * HW
                    i_v[4 * q + 0, s] = base00
                    i_v[4 * q + 1, s] = base00 + du
                    i_v[4 * q + 2, s] = base00 + dv
                    i_v[4 * q + 3, s] = base00 + dv + du
                    w_v[pl.ds((4 * q + 0) * B + i * L, L)] = (1.0 - wx) * (1.0 - wy)
                    w_v[pl.ds((4 * q + 1) * B + i * L, L)] = wx * (1.0 - wy)
                    w_v[pl.ds((4 * q + 2) * B + i * L, L)] = (1.0 - wx) * wy
                    w_v[pl.ds((4 * q + 3) * B + i * L, L)] = wx * wy
                    return carry2

                lax.fori_loop(0, B // L, iw_body, 0)
            for j in range(12):
                pltpu.async_copy(table2.at[i_v.at[j]], r_bufs[h][j], sem)

        def stage2(t, h):
            i_v, w_v, sem = i_bufs[h], w_bufs[h], sems[h]
            for q in range(3):
                for j in range(4 * q, 4 * q + 4):
                    pltpu.make_async_copy(
                        table2.at[i_v.at[j]], r_bufs[h][j], sem).wait()
                rr = [r_bufs[h][4 * q + k] for k in range(4)]
                t0 = dc_v[pl.ds(0, L)] + q * NT
                t1 = dc_v[pl.ds(L, L)] + q * NT
                u0 = sr_v[pl.ds(0, L)]

                def fma_body(g, carry2, q=q, rr=rr, t0=t0, t1=t1, u0=u0,
                             w_v=w_v):
                    w0 = w_v[pl.ds((4 * q + 0) * B + g * L, L)]
                    w1 = w_v[pl.ds((4 * q + 1) * B + g * L, L)]
                    w2 = w_v[pl.ds((4 * q + 2) * B + g * L, L)]
                    w3 = w_v[pl.ds((4 * q + 3) * B + g * L, L)]
                    for p in range(L):
                        pp = g * L + p
                        a0l = rr[0][pp, pl.ds(0, L)]
                        a0h = rr[0][pp, pl.ds(L, L)]
                        a1l = rr[1][pp, pl.ds(0, L)]
                        a1h = rr[1][pp, pl.ds(L, L)]
                        a2l = rr[2][pp, pl.ds(0, L)]
                        a2h = rr[2][pp, pl.ds(L, L)]
                        a3l = rr[3][pp, pl.ds(0, L)]
                        a3h = rr[3][pp, pl.ds(L, L)]
                        s0w, s1w, s2w, s3w = w0[p], w1[p], w2[p], w3[p]
                        lo = s0w * a0l + s1w * a1l + s2w * a2l + s3w * a3l
                        hi = s0w * a0h + s1w * a1h + s2w * a2h + s3w * a3h
                        sv = jnp.full((L,), pp, dtype=jnp.int32)
                        plsc.store_scatter(o_v, [t0, u0, sv], lo)
                        plsc.store_scatter(o_v, [t1, u0, sv], hi)
                    return carry2

                lax.fori_loop(0, B // L, fma_body, 0)

            pltpu.sync_copy(
                o_v.at[:, :, pl.ds(0, 128)],
                out_hbm.at[pl.ds(n_id * 3 * NT, 3 * NT),
                           (s0 + t * B) // 128, :, :])

        stage1(0, 0)
        stage1(1, 1)

        def pair_body(k, carry):
            stage2(2 * k, 0)
            stage1(jnp.minimum(2 * k + 2, NCHUNK - 1), 0)
            stage2(2 * k + 1, 1)
            stage1(jnp.minimum(2 * k + 3, NCHUNK - 1), 1)
            return carry

        lax.fori_loop(0, NCHUNK // 2, pair_body, 0)
        for h in range(2):
            for j in range(12):
                pltpu.make_async_copy(
                    table2.at[i_bufs[h].at[j]], r_bufs[h][j], sems[h]).wait()

    return tri


def kernel(x, fmx, fmy, fmz):
    N, S, _ = x.shape
    C = fmx.shape[1]
    H, W = fmx.shape[2], fmx.shape[3]
    P = N * S

    planes = jnp.stack([fmx[0], fmy[0], fmz[0]], axis=0)
    table = planes.transpose(0, 2, 3, 1).reshape(3 * H * W, C)
    xt = x.reshape(P, 3).T.reshape(3 * P)

    raw = _build(N, S, H, W, C)(xt, table)
    o = raw.reshape(N, 3, C // 8, S // 128, 8, 128)
    o = o.transpose(0, 3, 5, 1, 2, 4)
    return o.reshape(N, S, 3, C)
